# Initial kernel scaffold; baseline (speedup 1.0000x reference)
import jax
import jax.numpy as jnp
from jax.experimental import pallas as pl


def kernel(entity_emb, user_emb, latent_emb, edge_index, edge_type, interact_mat, weight, disen_weight_att):
    raise NotImplementedError("write your pallas kernel here")


# speedup vs baseline: 4.3493x; 4.3493x over previous
import functools

import jax
import jax.numpy as jnp
from jax import lax
from jax.experimental import pallas as pl
from jax.experimental.pallas import tpu as pltpu
from jax.experimental.pallas import tpu_sc as plsc

NSC = 2
NTILE = 16
NW = NSC * NTILE
LANES = 16
ACCW = 144
BATCH = 80


def _sc_body(n_ent, channel, batch, nbatch,
             entity_hbm, comb_hbm, weight_hbm, out_hbm,
             comb_v, weight_v, rows_v, staged_v, tail_i, head_i, acc_sh, sem):
    cid = lax.axis_index("c")
    sid = lax.axis_index("s")
    w = cid * NTILE + sid

    rows_per_tile = n_ent // NTILE
    nch = channel // LANES
    naccw = ACCW // LANES
    ngrp = batch // LANES

    zvec = jnp.zeros((LANES,), jnp.float32)

    def _zrow(b, _):
        for c in range(naccw):
            staged_v[b, pl.ds(c * LANES, LANES)] = zvec
        return 0

    lax.fori_loop(0, batch, _zrow, 0)

    nfull = rows_per_tile // batch
    rem = rows_per_tile - nfull * batch
    for z in range(nfull):
        pltpu.sync_copy(staged_v,
                        acc_sh.at[pl.ds(sid * rows_per_tile + z * batch, batch)])
    pltpu.sync_copy(staged_v.at[pl.ds(0, rem)],
                    acc_sh.at[pl.ds(sid * rows_per_tile + nfull * batch, rem)])

    pltpu.sync_copy(comb_hbm.at[w], comb_v)
    pltpu.sync_copy(weight_hbm, weight_v)

    onevec = jnp.where(lax.iota(jnp.int32, LANES) == 0,
                       jnp.float32(1.0), jnp.float32(0.0))

    def _ones(b, _):
        staged_v[b, pl.ds(channel, LANES)] = onevec
        return 0

    lax.fori_loop(0, batch, _ones, 0)

    plsc.subcore_barrier()

    mask14 = jnp.int32(0x3FFF)

    def _batch(j, _):
        def _unpack(g, carry):
            cvec = comb_v[j, pl.ds(g * LANES, LANES)]
            tail_i[pl.ds(g * LANES, LANES)] = (
                lax.shift_right_logical(cvec, 14) & mask14)
            head_i[pl.ds(g * LANES, LANES)] = cvec & mask14
            return carry

        lax.fori_loop(0, ngrp, _unpack, 0)

        pltpu.async_copy(entity_hbm.at[tail_i], rows_v, sem).wait()

        def _group(g, carry):
            cvec = comb_v[j, pl.ds(g * LANES, LANES)]
            rvec = (lax.shift_right_logical(cvec, 28) & 3) * channel
            for l in range(LANES):
                roff = rvec[l]
                b = g * LANES + l
                for c in range(nch):
                    x = rows_v[b, pl.ds(c * LANES, LANES)]
                    wv = weight_v[pl.ds(roff + c * LANES, LANES)]
                    staged_v[b, pl.ds(c * LANES, LANES)] = x * wv
            return carry

        lax.fori_loop(0, ngrp, _group, 0)

        pltpu.sync_copy(staged_v, acc_sh.at[head_i], add=True)
        return 0

    lax.fori_loop(0, nbatch, _batch, 0)

    plsc.subcore_barrier()

    pltpu.sync_copy(acc_sh.at[pl.ds(sid * rows_per_tile, rows_per_tile)],
                    out_hbm.at[cid, pl.ds(sid * rows_per_tile, rows_per_tile)])


def _entity_finalize_body(p_ref, out_ref):
    s = p_ref[0] + p_ref[1]
    cnt = jnp.maximum(s[:, 128:129], 1.0)
    out_ref[...] = s[:, :128] / cnt


def _user_body(im_ref, ent_ref, ue_ref, le_ref, w_ref, da_ref, out_ref):
    acc = jnp.dot(im_ref[...], ent_ref[...], preferred_element_type=jnp.float32)
    score_ = lax.dot_general(ue_ref[...], le_ref[...],
                             (((1,), (1,)), ((), ())),
                             preferred_element_type=jnp.float32)
    score = jax.nn.softmax(score_, axis=1)
    dw = jnp.dot(jax.nn.softmax(da_ref[...], axis=-1), w_ref[...],
                 preferred_element_type=jnp.float32)
    coef = jnp.dot(score, dw, preferred_element_type=jnp.float32)
    out_ref[...] = acc * (1.0 + coef)


def kernel(entity_emb, user_emb, latent_emb, edge_index, edge_type,
           interact_mat, weight, disen_weight_att):
    n_ent, channel = entity_emb.shape
    n_users = user_emb.shape[0]
    n_rel = weight.shape[0]
    n_edges = edge_index.shape[1]

    e_per_tile = n_edges // NW
    nbatch = e_per_tile // BATCH

    head = edge_index[0].astype(jnp.int32)
    tail = edge_index[1].astype(jnp.int32)
    rel = (edge_type.astype(jnp.int32) - 1) % n_rel
    comb = (head | (tail << 14) | (rel << 28)).reshape(NW, nbatch, BATCH)
    weight_flat = weight.reshape(-1)
    comb, weight_flat = lax.optimization_barrier((comb, weight_flat))

    mesh = plsc.VectorSubcoreMesh(core_axis_name="c", subcore_axis_name="s")
    sc_call = pl.kernel(
        functools.partial(_sc_body, n_ent, channel, BATCH, nbatch),
        out_type=jax.ShapeDtypeStruct((NSC, n_ent, ACCW), jnp.float32),
        mesh=mesh,
        compiler_params=pltpu.CompilerParams(use_tc_tiling_on_sc=False),
        scratch_types=[
            pltpu.VMEM((nbatch, BATCH), jnp.int32),
            pltpu.VMEM((n_rel * channel,), jnp.float32),
            pltpu.VMEM((BATCH, channel), jnp.float32),
            pltpu.VMEM((BATCH, ACCW), jnp.float32),
            pltpu.VMEM((BATCH,), jnp.int32),
            pltpu.VMEM((BATCH,), jnp.int32),
            pltpu.VMEM_SHARED((n_ent, ACCW), jnp.float32),
            pltpu.SemaphoreType.DMA,
        ],
    )
    partials = sc_call(entity_emb, comb, weight_flat)

    entity_agg = pl.pallas_call(
        _entity_finalize_body,
        out_shape=jax.ShapeDtypeStruct((n_ent, channel), jnp.float32),
    )(partials)

    bm = 256
    user_agg = pl.pallas_call(
        _user_body,
        grid=(n_users // bm,),
        in_specs=[
            pl.BlockSpec((bm, n_ent), lambda i: (i, 0)),
            pl.BlockSpec((n_ent, channel), lambda i: (0, 0)),
            pl.BlockSpec((bm, channel), lambda i: (i, 0)),
            pl.BlockSpec(latent_emb.shape, lambda i: (0, 0)),
            pl.BlockSpec(weight.shape, lambda i: (0, 0)),
            pl.BlockSpec(disen_weight_att.shape, lambda i: (0, 0)),
        ],
        out_specs=pl.BlockSpec((bm, channel), lambda i: (i, 0)),
        out_shape=jax.ShapeDtypeStruct((n_users, channel), jnp.float32),
    )(interact_mat, entity_emb, user_emb, latent_emb, weight,
      disen_weight_att)

    return (entity_agg, user_agg)
